# baseline (speedup 1.0000x reference)
import functools

import jax
import jax.numpy as jnp
from jax import lax
from jax.experimental import pallas as pl
from jax.experimental.pallas import tpu as pltpu
from jax.experimental.pallas import tpu_sc as plsc

D = 64
NCOMP = 4
LANES = 16

NC = 2
NS = 16
NW = NC * NS


@jax.jit
def _sc_embed(idx_t, Wt_r, Wt_i, Wt_j, Wt_k):
    hist, batch = idx_t.shape
    vocab = Wt_r.shape[1]
    d_per_w = D // NW
    n_vec = batch // LANES

    mesh = plsc.VectorSubcoreMesh(core_axis_name="c", subcore_axis_name="s")

    @functools.partial(
        pl.kernel,
        mesh=mesh,
        compiler_params=pltpu.CompilerParams(
            needs_layout_passes=False, use_tc_tiling_on_sc=True),
        out_type=jax.ShapeDtypeStruct((hist, D, NCOMP, batch), jnp.float32),
        scratch_types=[
            pltpu.VMEM_SHARED((hist * batch,), jnp.int32),
            pltpu.VMEM((vocab,), jnp.float32),
            pltpu.VMEM((2, batch), jnp.int32),
            pltpu.VMEM((2, batch), jnp.float32),
            pltpu.SemaphoreType.DMA,
            pltpu.SemaphoreType.DMA,
            pltpu.SemaphoreType.DMA,
            pltpu.SemaphoreType.DMA,
        ],
    )
    def kern(idx_hbm, wr_hbm, wi_hbm, wj_hbm, wk_hbm, out_hbm,
             sidx, row_v, idxr, obuf, xsem0, xsem1, osem0, osem1):
        xsem = (xsem0, xsem1)
        osem = (osem0, osem1)
        cid = lax.axis_index("c")
        sid = lax.axis_index("s")
        wid = sid * NC + cid

        for r in range((hist + NS - 1) // NS):
            h0 = sid + NS * r

            @pl.when(h0 < hist)
            def _():
                pltpu.sync_copy(idx_hbm.at[h0],
                                sidx.at[pl.ds(h0 * batch, batch)])

        plsc.subcore_barrier()

        UNROLL = 16

        for dd in range(d_per_w):
            d = wid * d_per_w + dd
            for c, w in enumerate((wr_hbm, wi_hbm, wj_hbm, wk_hbm)):
                pltpu.sync_copy(w.at[d], row_v)
                pltpu.async_copy(sidx.at[pl.ds(0, batch)], idxr.at[0], xsem[0])
                pltpu.async_copy(sidx.at[pl.ds(batch, batch)], idxr.at[1], xsem[1])

                @pl.loop(0, hist // 2)
                def _(hh):
                    for p in range(2):
                        h = 2 * hh + p

                        pltpu.make_async_copy(
                            sidx.at[pl.ds(h * batch, batch)], idxr.at[p], xsem[p]).wait()

                        @pl.when(h >= 2)
                        def _():
                            pltpu.make_async_copy(
                                obuf.at[p], out_hbm.at[h - 2, d, c],
                                osem[p]).wait()

                        @pl.loop(0, n_vec // UNROLL)
                        def _(i):
                            offs = [(i * UNROLL + u) * LANES
                                    for u in range(UNROLL)]
                            ivs = [idxr[p, pl.ds(off, LANES)]
                                   for off in offs]
                            vals = [plsc.load_gather(row_v, [iv])
                                    for iv in ivs]
                            for off, val in zip(offs, vals):
                                obuf[p, pl.ds(off, LANES)] = val

                        pltpu.async_copy(
                            obuf.at[p], out_hbm.at[h, d, c], osem[p])

                        @pl.when(h + 2 < hist)
                        def _():
                            pltpu.async_copy(
                                sidx.at[pl.ds((h + 2) * batch, batch)],
                                idxr.at[p], xsem[p])

                for hh in (hist - 2, hist - 1):
                    pltpu.make_async_copy(
                        obuf.at[hh % 2], out_hbm.at[hh, d, c],
                        osem[hh % 2]).wait()

    return kern(idx_t, Wt_r, Wt_i, Wt_j, Wt_k)


def kernel(input_ids, W_r, W_i, W_j, W_k):
    batch, hist = input_ids.shape
    idx_t = input_ids.T
    out_t = _sc_embed(idx_t, W_r.T, W_i.T, W_j.T, W_k.T)
    return out_t.transpose(3, 0, 1, 2)
